# baseline (speedup 1.0000x reference)
import functools

import jax
import jax.numpy as jnp
from jax import lax
from jax.experimental import pallas as pl
from jax.experimental.pallas import tpu as pltpu
from jax.experimental.pallas import tpu_sc as plsc

_VOCAB = 30522
_D = 128
_B = 16384
_CH = 128


@functools.lru_cache(maxsize=1)
def _build():
    info = plsc.get_sparse_core_info()
    nw = info.num_cores * info.num_subcores
    b_per_w = _B // nw
    nch = b_per_w // _CH
    units = 2 * nch
    nbuf = 4
    mesh = plsc.VectorSubcoreMesh(core_axis_name="c", subcore_axis_name="s")
    out_sds = jax.ShapeDtypeStruct((_B, _D), jnp.float32)

    @functools.partial(
        pl.kernel,
        mesh=mesh,
        out_type=[out_sds, out_sds],
        scratch_types=[
            pltpu.VMEM((units, _CH), jnp.int32),
            pltpu.VMEM((nbuf * _CH, _D), jnp.float32),
        ] + [pltpu.SemaphoreType.DMA] * (2 * nbuf),
    )
    def sc_gather2(go_idx, rel_idx, go_tab, rel_tab, go_out, rel_out,
                   idx_v, rows_v, *sems):
        gsem, osem = sems[:nbuf], sems[nbuf:]
        wid = lax.axis_index("s") * info.num_cores + lax.axis_index("c")
        base = wid * nch

        pltpu.sync_copy(go_idx.at[pl.ds(base, nch)], idx_v.at[pl.ds(0, nch)])
        pltpu.sync_copy(rel_idx.at[pl.ds(base, nch)], idx_v.at[pl.ds(nch, nch)])

        tabs = [go_tab] * nch + [rel_tab] * nch
        outs = [go_out] * nch + [rel_out] * nch

        def buf(u):
            return rows_v.at[pl.ds((u % nbuf) * _CH, _CH)]

        def out_slice(u):
            return outs[u].at[pl.ds(wid * b_per_w + (u % nch) * _CH, _CH)]

        gcp = [None] * units
        ocp = [None] * units
        for step in range(units + 2):
            if step < units:
                if step >= nbuf:
                    ocp[step - nbuf].wait()
                gcp[step] = pltpu.async_copy(
                    tabs[step].at[idx_v.at[step]], buf(step),
                    gsem[step % nbuf])
            if step >= 2:
                u = step - 2
                gcp[u].wait()
                ocp[u] = pltpu.async_copy(buf(u), out_slice(u),
                                          osem[u % nbuf])
        for u in range(units - nbuf, units):
            ocp[u].wait()

    return sc_gather2


def kernel(go_inputs, relation_ids, go_table, rel_table):
    k = _build()
    go_idx = go_inputs.astype(jnp.int32).reshape(_B // _CH, _CH)
    rel_idx = relation_ids.astype(jnp.int32).reshape(_B // _CH, _CH)
    entity_embed, relation_embed = k(go_idx, rel_idx, go_table, rel_table)
    return (entity_embed, relation_embed)


# speedup vs baseline: 1.0241x; 1.0241x over previous
import functools

import jax
import jax.numpy as jnp
from jax import lax
from jax.experimental import pallas as pl
from jax.experimental.pallas import tpu as pltpu
from jax.experimental.pallas import tpu_sc as plsc

_VOCAB = 30522
_D = 128
_B = 16384
_CH = 128


@functools.lru_cache(maxsize=1)
def _build():
    info = plsc.get_sparse_core_info()
    nw = info.num_cores * info.num_subcores
    b_per_w = _B // nw
    nch = b_per_w // _CH
    units = 2 * nch
    nbuf = 7
    mesh = plsc.VectorSubcoreMesh(core_axis_name="c", subcore_axis_name="s")
    out_sds = jax.ShapeDtypeStruct((_B, _D), jnp.float32)

    @functools.partial(
        pl.kernel,
        mesh=mesh,
        out_type=[out_sds, out_sds],
        scratch_types=[
            pltpu.VMEM((units, _CH), jnp.int32),
            pltpu.VMEM((nbuf * _CH, _D), jnp.float32),
        ] + [pltpu.SemaphoreType.DMA] * (2 * nbuf),
    )
    def sc_gather2(go_idx, rel_idx, go_tab, rel_tab, go_out, rel_out,
                   idx_v, rows_v, *sems):
        gsem, osem = sems[:nbuf], sems[nbuf:]
        wid = lax.axis_index("s") * info.num_cores + lax.axis_index("c")
        base = wid * nch

        pltpu.sync_copy(go_idx.at[pl.ds(base, nch)], idx_v.at[pl.ds(0, nch)])
        pltpu.sync_copy(rel_idx.at[pl.ds(base, nch)], idx_v.at[pl.ds(nch, nch)])

        tabs = [go_tab] * nch + [rel_tab] * nch
        outs = [go_out] * nch + [rel_out] * nch

        def buf(u):
            return rows_v.at[pl.ds((u % nbuf) * _CH, _CH)]

        def out_slice(u):
            return outs[u].at[pl.ds(wid * b_per_w + (u % nch) * _CH, _CH)]

        gcp = [None] * units
        ocp = [None] * units
        for u in range(min(nbuf, units)):
            gcp[u] = pltpu.async_copy(
                tabs[u].at[idx_v.at[u]], buf(u), gsem[u % nbuf])
        for u in range(units):
            gcp[u].wait()
            ocp[u] = pltpu.async_copy(buf(u), out_slice(u), osem[u % nbuf])
            refire = u + nbuf
            if refire < units:
                ocp[refire - nbuf].wait()
                gcp[refire] = pltpu.async_copy(
                    tabs[refire].at[idx_v.at[refire]], buf(refire),
                    gsem[refire % nbuf])
        for u in range(max(0, units - nbuf), units):
            ocp[u].wait()

    return sc_gather2


def kernel(go_inputs, relation_ids, go_table, rel_table):
    k = _build()
    go_idx = go_inputs.astype(jnp.int32).reshape(_B // _CH, _CH)
    rel_idx = relation_ids.astype(jnp.int32).reshape(_B // _CH, _CH)
    entity_embed, relation_embed = k(go_idx, rel_idx, go_table, rel_table)
    return (entity_embed, relation_embed)


# speedup vs baseline: 1.0393x; 1.0148x over previous
import functools

import jax
import jax.numpy as jnp
from jax import lax
from jax.experimental import pallas as pl
from jax.experimental.pallas import tpu as pltpu
from jax.experimental.pallas import tpu_sc as plsc

_VOCAB = 30522
_D = 128
_B = 16384
_CH = 128


@functools.lru_cache(maxsize=1)
def _build():
    info = plsc.get_sparse_core_info()
    nw = info.num_cores * info.num_subcores
    b_per_w = _B // nw
    nch = b_per_w // _CH
    units = 2 * nch
    nbuf = 7
    mesh = plsc.VectorSubcoreMesh(core_axis_name="c", subcore_axis_name="s")
    out_sds = jax.ShapeDtypeStruct((_B, _D), jnp.float32)

    @functools.partial(
        pl.kernel,
        mesh=mesh,
        out_type=[out_sds, out_sds],
        scratch_types=[
            pltpu.VMEM((units, _CH), jnp.int32),
            pltpu.VMEM((nbuf * _CH, _D), jnp.float32),
        ] + [pltpu.SemaphoreType.DMA] * (2 * nbuf),
    )
    def sc_gather2(idx_all, go_tab, rel_tab, go_out, rel_out,
                   idx_v, rows_v, *sems):
        gsem, osem = sems[:nbuf], sems[nbuf:]
        wid = lax.axis_index("s") * info.num_cores + lax.axis_index("c")

        pltpu.sync_copy(idx_all.at[wid], idx_v)

        tabs = [go_tab] * nch + [rel_tab] * nch
        outs = [go_out] * nch + [rel_out] * nch

        def buf(u):
            return rows_v.at[pl.ds((u % nbuf) * _CH, _CH)]

        def out_slice(u):
            return outs[u].at[pl.ds(wid * b_per_w + (u % nch) * _CH, _CH)]

        gcp = [None] * units
        ocp = [None] * units
        for u in range(min(nbuf, units)):
            gcp[u] = pltpu.async_copy(
                tabs[u].at[idx_v.at[u]], buf(u), gsem[u % nbuf])
        for u in range(units):
            gcp[u].wait()
            ocp[u] = pltpu.async_copy(buf(u), out_slice(u), osem[u % nbuf])
            refire = u + nbuf
            if refire < units:
                ocp[u].wait()
                gcp[refire] = pltpu.async_copy(
                    tabs[refire].at[idx_v.at[refire]], buf(refire),
                    gsem[refire % nbuf])
        for u in range(max(0, units - nbuf), units):
            ocp[u].wait()

    return sc_gather2, nw, nch


def kernel(go_inputs, relation_ids, go_table, rel_table):
    k, nw, nch = _build()
    go_idx = go_inputs.reshape(nw, nch, _CH)
    rel_idx = relation_ids.reshape(nw, nch, _CH)
    idx_all = jnp.concatenate([go_idx, rel_idx], axis=1)
    entity_embed, relation_embed = k(idx_all, go_table, rel_table)
    return (entity_embed, relation_embed)
